# baseline (speedup 1.0000x reference)
import functools

import numpy as np
import jax
import jax.numpy as jnp
from jax import lax
from jax.experimental import pallas as pl
from jax.experimental.pallas import tpu as pltpu
from jax.experimental.pallas import tpu_sc as plsc

_BETA = 0.9
_TAU = 1.0 / (1.0 - _BETA)
_INV_TAU = np.float32(1.0 / _TAU)
_THRESHOLD = np.float32(1.0)
_NUM_IN = 8
_NUM_HID = 8
_NUM_OUT = 4
_STEPS = 16
_L = 16

_LCG_A = np.uint32(747796405)
_LCG_C = np.uint32(2891336453)
_SEED_M = np.uint32(2654435761)
_SEED_C = np.uint32(0x9E3779B9)


def _sc_body(ncores, chunk, sc_base, x_hbm, ns_hbm, out_hbm, xv, ov, nsv, sem):
    cid = lax.axis_index("c")
    sid = lax.axis_index("s")
    wid = sid * ncores + cid
    base = sc_base + wid * chunk

    copies = [
        pltpu.make_async_copy(
            x_hbm.at[c, pl.ds(base, chunk)],
            xv.at[pl.ds(c * chunk, chunk)],
            sem,
        )
        for c in range(_NUM_IN)
    ]
    for cp in copies:
        cp.start()
    pltpu.sync_copy(ns_hbm, nsv)
    for cp in copies:
        cp.wait()

    iota = lax.iota(jnp.int32, _L)
    ones = jnp.full((_L,), 1.0, jnp.float32)
    zeros = jnp.full((_L,), 0.0, jnp.float32)
    ns_i = nsv[...]

    actives = [
        jnp.where(jnp.full((_L,), t, jnp.int32) < ns_i, ones, zeros)
        for t in range(_STEPS)
    ]

    gidx = jnp.full((_L,), base, jnp.int32) + iota
    seeds0 = []
    for p in range(_NUM_IN):
        s = (gidx * np.int32(_NUM_IN) + np.int32(p)).astype(jnp.uint32)
        s = s * _SEED_M + _SEED_C
        s = s ^ (s >> np.uint32(16))
        s = s * _LCG_A + _LCG_C
        seeds0.append(s)

    num_groups = chunk // _L

    def sim_group(off, rng):
        thr = [
            (xv[pl.ds(c * chunk + off, _L)] * np.float32(1 << 16))
            .astype(jnp.int32)
            for c in range(_NUM_IN)
        ]

        rng = list(rng)
        vh = [zeros] * _NUM_HID
        vo = [zeros] * _NUM_OUT
        acc = [zeros] * _NUM_OUT
        for t in range(_STEPS):
            active = actives[t]
            spikes_in = [None] * _NUM_IN
            for p in range(_NUM_IN // 2):
                s = rng[p] * _LCG_A + _LCG_C
                u_hi = plsc.bitcast(s >> np.uint32(16), jnp.int32)
                u_lo = plsc.bitcast(s & np.uint32(0xFFFF), jnp.int32)
                spikes_in[p] = jnp.where(u_hi < thr[p], ones, zeros)
                spikes_in[p + 4] = jnp.where(u_lo < thr[p + 4], ones, zeros)
                rng[p] = s
            hs = []
            for h in range(_NUM_HID):
                agg = spikes_in[h] - spikes_in[(h + 3) % _NUM_IN]
                v_new = vh[h] + (agg - vh[h]) * _INV_TAU
                fired = v_new >= _THRESHOLD
                hs.append(jnp.where(fired, ones, zeros))
                vh[h] = jnp.where(fired, zeros, v_new)
            q = [hs[2 * e] + hs[2 * e + 1] for e in range(_NUM_OUT)]
            for o in range(_NUM_OUT):
                agg = q[o] - q[(o + 2) % _NUM_OUT]
                v_new = vo[o] + (agg - vo[o]) * _INV_TAU
                fired = v_new >= _THRESHOLD
                vo[o] = jnp.where(fired, zeros, v_new)
                acc[o] = acc[o] + jnp.where(fired, active, zeros)
        for o in range(_NUM_OUT):
            ov[pl.ds(o * chunk + off, _L)] = acc[o]
        return tuple(rng)

    def group_pair_body(gp, rng):
        off = gp * np.int32(2 * _L)
        rng_a = sim_group(off, rng[: _NUM_IN // 2])
        rng_b = sim_group(off + np.int32(_L), rng[_NUM_IN // 2:])
        return rng_a + rng_b

    lax.fori_loop(0, num_groups // 2, group_pair_body, tuple(seeds0))

    out_copies = [
        pltpu.make_async_copy(
            ov.at[pl.ds(o * chunk, chunk)],
            out_hbm.at[o, pl.ds(base - sc_base, chunk)],
            sem,
        )
        for o in range(_NUM_OUT)
    ]
    for cp in out_copies:
        cp.start()
    for cp in out_copies:
        cp.wait()


def _tc_body(ns_ref, x_ref, out_ref):
    bt = x_ref.shape[1]
    pltpu.prng_seed(0x5CBA17)
    thr = (x_ref[...] * np.float32(1 << 24)).astype(jnp.int32)
    ns = ns_ref[0]
    zero = np.float32(0.0)
    one = np.float32(1.0)
    vh = jnp.zeros((_NUM_HID, bt), jnp.float32)
    vo = jnp.zeros((_NUM_HID, bt), jnp.float32)
    acc = jnp.zeros((_NUM_HID, bt), jnp.float32)
    for t in range(_STEPS):
        bits = pltpu.prng_random_bits((_NUM_IN, bt)).astype(jnp.uint32)
        u = (bits >> np.uint32(8)).astype(jnp.int32)
        spikes = jnp.where(u < thr, one, zero)
        aggh = spikes - jnp.concatenate([spikes[3:], spikes[:3]], axis=0)
        v_new = vh + (aggh - vh) * _INV_TAU
        firedh = v_new >= _THRESHOLD
        hsp = jnp.where(firedh, one, zero)
        vh = jnp.where(firedh, zero, v_new)
        q = hsp + jnp.concatenate([hsp[1:], hsp[:1]], axis=0)
        aggo = q - jnp.concatenate([q[4:], q[:4]], axis=0)
        v_new_o = vo + (aggo - vo) * _INV_TAU
        firedo = v_new_o >= _THRESHOLD
        vo = jnp.where(firedo, zero, v_new_o)
        active = jnp.where(t < ns, one, zero)
        acc = acc + jnp.where(firedo, active, zero)
    for o in range(_NUM_OUT):
        out_ref[o, :] = acc[2 * o, :]


def kernel(x, num_steps):
    batch = x.shape[0]
    num_cores = 1
    num_workers = num_cores * 16
    b_tc = (batch * 7) // 8
    b_sc = batch - b_tc
    assert b_sc % (num_workers * _L) == 0 and b_tc % 128 == 0
    chunk = b_sc // num_workers
    ns_arr = jnp.full((_L,), num_steps, dtype=jnp.int32)
    xt = x.T

    mesh = plsc.VectorSubcoreMesh(
        core_axis_name="c", subcore_axis_name="s", num_cores=num_cores)
    run_sc = pl.kernel(
        functools.partial(_sc_body, num_cores, chunk, b_tc),
        out_type=jax.ShapeDtypeStruct((_NUM_OUT, b_sc), jnp.float32),
        mesh=mesh,
        compiler_params=pltpu.CompilerParams(needs_layout_passes=False),
        scratch_types=[
            pltpu.VMEM((chunk * _NUM_IN,), jnp.float32),
            pltpu.VMEM((chunk * _NUM_OUT,), jnp.float32),
            pltpu.VMEM((_L,), jnp.int32),
            pltpu.SemaphoreType.DMA,
        ],
    )
    sc_out = run_sc(xt, ns_arr)

    tc_out = pl.pallas_call(
        _tc_body,
        out_shape=jax.ShapeDtypeStruct((_NUM_OUT, b_tc), jnp.float32),
        grid=(1,),
        in_specs=[
            pl.BlockSpec(memory_space=pltpu.SMEM),
            pl.BlockSpec((_NUM_IN, b_tc), lambda i: (0, 0)),
        ],
        out_specs=pl.BlockSpec((_NUM_OUT, b_tc), lambda i: (0, 0)),
    )(ns_arr, xt)

    out_t = jnp.concatenate([tc_out, sc_out], axis=1)
    return out_t.T


# speedup vs baseline: 1.0534x; 1.0534x over previous
import functools

import numpy as np
import jax
import jax.numpy as jnp
from jax import lax
from jax.experimental import pallas as pl
from jax.experimental.pallas import tpu as pltpu
from jax.experimental.pallas import tpu_sc as plsc

_BETA = 0.9
_TAU = 1.0 / (1.0 - _BETA)
_INV_TAU = np.float32(1.0 / _TAU)
_THRESHOLD = np.float32(1.0)
_NUM_IN = 8
_NUM_HID = 8
_NUM_OUT = 4
_STEPS = 16
_L = 16

_LCG_A = np.uint32(747796405)
_LCG_C = np.uint32(2891336453)
_SEED_M = np.uint32(2654435761)
_SEED_C = np.uint32(0x9E3779B9)


def _sc_body(ncores, chunk, sc_base, x_hbm, ns_hbm, out_hbm, xv, ov, nsv, sem):
    cid = lax.axis_index("c")
    sid = lax.axis_index("s")
    wid = sid * ncores + cid
    base = sc_base + wid * chunk

    copies = [
        pltpu.make_async_copy(
            x_hbm.at[c, pl.ds(base, chunk)],
            xv.at[pl.ds(c * chunk, chunk)],
            sem,
        )
        for c in range(_NUM_IN)
    ]
    for cp in copies:
        cp.start()
    pltpu.sync_copy(ns_hbm, nsv)
    for cp in copies:
        cp.wait()

    iota = lax.iota(jnp.int32, _L)
    ones = jnp.full((_L,), 1.0, jnp.float32)
    zeros = jnp.full((_L,), 0.0, jnp.float32)
    ns_i = nsv[...]

    actives = [
        jnp.where(jnp.full((_L,), t, jnp.int32) < ns_i, ones, zeros)
        for t in range(_STEPS)
    ]

    gidx = jnp.full((_L,), base, jnp.int32) + iota
    seeds0 = []
    for p in range(_NUM_IN // 2):
        s = (gidx * np.int32(_NUM_IN // 2) + np.int32(p)).astype(jnp.uint32)
        s = s * _SEED_M + _SEED_C
        s = s ^ (s >> np.uint32(16))
        s = s * _LCG_A + _LCG_C
        seeds0.append(s)

    num_groups = chunk // _L

    def sim_group(off, rng):
        thr = [
            (xv[pl.ds(c * chunk + off, _L)] * np.float32(1 << 16))
            .astype(jnp.int32)
            for c in range(_NUM_IN)
        ]

        rng = list(rng)
        vh = [zeros] * _NUM_HID
        vo = [zeros] * _NUM_OUT
        acc = [zeros] * _NUM_OUT
        for t in range(_STEPS):
            active = actives[t]
            spikes_in = [None] * _NUM_IN
            for p in range(_NUM_IN // 2):
                s = rng[p] * _LCG_A + _LCG_C
                u_hi = plsc.bitcast(s >> np.uint32(16), jnp.int32)
                u_lo = plsc.bitcast(s & np.uint32(0xFFFF), jnp.int32)
                spikes_in[p] = jnp.where(u_hi < thr[p], ones, zeros)
                spikes_in[p + 4] = jnp.where(u_lo < thr[p + 4], ones, zeros)
                rng[p] = s
            hs = []
            for h in range(_NUM_HID):
                agg = spikes_in[h] - spikes_in[(h + 3) % _NUM_IN]
                v_new = vh[h] + (agg - vh[h]) * _INV_TAU
                fired = v_new >= _THRESHOLD
                hs.append(jnp.where(fired, ones, zeros))
                vh[h] = jnp.where(fired, zeros, v_new)
            q = [hs[2 * e] + hs[2 * e + 1] for e in range(_NUM_OUT)]
            for o in range(_NUM_OUT):
                agg = q[o] - q[(o + 2) % _NUM_OUT]
                v_new = vo[o] + (agg - vo[o]) * _INV_TAU
                fired = v_new >= _THRESHOLD
                vo[o] = jnp.where(fired, zeros, v_new)
                acc[o] = acc[o] + jnp.where(fired, active, zeros)
        for o in range(_NUM_OUT):
            ov[pl.ds(o * chunk + off, _L)] = acc[o]
        return tuple(rng)

    def group_body(g, rng):
        return sim_group(g * np.int32(_L), rng)

    lax.fori_loop(0, num_groups, group_body, tuple(seeds0))

    out_copies = [
        pltpu.make_async_copy(
            ov.at[pl.ds(o * chunk, chunk)],
            out_hbm.at[o, pl.ds(base - sc_base, chunk)],
            sem,
        )
        for o in range(_NUM_OUT)
    ]
    for cp in out_copies:
        cp.start()
    for cp in out_copies:
        cp.wait()


def _tc_body(ns_ref, x_ref, out_ref):
    bt = x_ref.shape[1]
    pltpu.prng_seed(0x5CBA17)
    thr = (x_ref[...] * np.float32(1 << 24)).astype(jnp.int32)
    ns = ns_ref[0]
    zero = np.float32(0.0)
    one = np.float32(1.0)
    vh = jnp.zeros((_NUM_HID, bt), jnp.float32)
    vo = jnp.zeros((_NUM_HID, bt), jnp.float32)
    acc = jnp.zeros((_NUM_HID, bt), jnp.float32)
    for t in range(_STEPS):
        bits = pltpu.prng_random_bits((_NUM_IN, bt)).astype(jnp.uint32)
        u = (bits >> np.uint32(8)).astype(jnp.int32)
        spikes = jnp.where(u < thr, one, zero)
        aggh = spikes - jnp.concatenate([spikes[3:], spikes[:3]], axis=0)
        v_new = vh + (aggh - vh) * _INV_TAU
        firedh = v_new >= _THRESHOLD
        hsp = jnp.where(firedh, one, zero)
        vh = jnp.where(firedh, zero, v_new)
        q = hsp + jnp.concatenate([hsp[1:], hsp[:1]], axis=0)
        aggo = q - jnp.concatenate([q[4:], q[:4]], axis=0)
        v_new_o = vo + (aggo - vo) * _INV_TAU
        firedo = v_new_o >= _THRESHOLD
        vo = jnp.where(firedo, zero, v_new_o)
        active = jnp.where(t < ns, one, zero)
        acc = acc + jnp.where(firedo, active, zero)
    for o in range(_NUM_OUT):
        out_ref[o, :] = acc[2 * o, :]


def kernel(x, num_steps):
    batch = x.shape[0]
    num_cores = 1
    num_workers = num_cores * 16
    b_tc = (batch * 7) // 8
    b_sc = batch - b_tc
    assert b_sc % (num_workers * _L) == 0 and b_tc % 128 == 0
    chunk = b_sc // num_workers
    ns_arr = jnp.full((_L,), num_steps, dtype=jnp.int32)
    xt = x.T

    mesh = plsc.VectorSubcoreMesh(
        core_axis_name="c", subcore_axis_name="s", num_cores=num_cores)
    run_sc = pl.kernel(
        functools.partial(_sc_body, num_cores, chunk, b_tc),
        out_type=jax.ShapeDtypeStruct((_NUM_OUT, b_sc), jnp.float32),
        mesh=mesh,
        compiler_params=pltpu.CompilerParams(needs_layout_passes=False),
        scratch_types=[
            pltpu.VMEM((chunk * _NUM_IN,), jnp.float32),
            pltpu.VMEM((chunk * _NUM_OUT,), jnp.float32),
            pltpu.VMEM((_L,), jnp.int32),
            pltpu.SemaphoreType.DMA,
        ],
    )
    sc_out = run_sc(xt, ns_arr)

    tc_out = pl.pallas_call(
        _tc_body,
        out_shape=jax.ShapeDtypeStruct((_NUM_OUT, b_tc), jnp.float32),
        grid=(1,),
        in_specs=[
            pl.BlockSpec(memory_space=pltpu.SMEM),
            pl.BlockSpec((_NUM_IN, b_tc), lambda i: (0, 0)),
        ],
        out_specs=pl.BlockSpec((_NUM_OUT, b_tc), lambda i: (0, 0)),
    )(ns_arr, xt)

    out_t = jnp.concatenate([tc_out, sc_out], axis=1)
    return out_t.T
